# baseline (speedup 1.0000x reference)
import math

import jax
import jax.numpy as jnp
from jax import lax
from jax.experimental import pallas as pl
from jax.experimental.pallas import tpu as pltpu
from jax.experimental.pallas import tpu_sc as plsc

D = 64
SCALE = math.sqrt(D)
NC = 2
NS = 16
NW = NC * NS
C = 128
NBUF = 4
LANES = 16


def _emb_body(xe_hbm, xo_hbm, table_hbm, out_hbm, idx_v, ge_v, go_v, pk_v,
              *sems):
    sem_e = sems[:NBUF]
    sem_o = sems[NBUF:2 * NBUF]
    sem_s = sems[2 * NBUF:]
    n_chunks = xe_hbm.shape[0] // NW
    wid = lax.axis_index("s") * NC + lax.axis_index("c")
    ibase = wid * n_chunks
    obase = wid * n_chunks * (C // 2)

    def gather_start(g, b):
        pltpu.make_async_copy(
            table_hbm.at[idx_v.at[0, g]], ge_v.at[b], sem_e[b]).start()
        pltpu.make_async_copy(
            table_hbm.at[idx_v.at[1, g]], go_v.at[b], sem_o[b]).start()

    def gather_wait(g, b):
        pltpu.make_async_copy(
            table_hbm.at[idx_v.at[0, g]], ge_v.at[b], sem_e[b]).wait()
        pltpu.make_async_copy(
            table_hbm.at[idx_v.at[1, g]], go_v.at[b], sem_o[b]).wait()

    def scatter(g, b, sem):
        return pltpu.make_async_copy(
            pk_v.at[b], out_hbm.at[pl.ds(obase + g * (C // 2), C // 2)],
            sem)

    pltpu.sync_copy(xe_hbm.at[pl.ds(ibase, n_chunks)], idx_v.at[0])
    pltpu.sync_copy(xo_hbm.at[pl.ds(ibase, n_chunks)], idx_v.at[1])

    for b in range(NBUF):
        gather_start(b, b)

    @pl.loop(0, n_chunks // NBUF)
    def _outer(t):
        g0 = t * NBUF
        for bb in range(NBUF):
            g = g0 + bb
            pb = (bb - 1) % NBUF
            p = g - 1
            nxt = p + NBUF

            @pl.when(jnp.logical_and(p >= 0, nxt < n_chunks))
            def _recycle(pb=pb, p=p, nxt=nxt):
                scatter(p, pb, sem_s[pb]).wait()
                gather_start(nxt, pb)

            gather_wait(g, bb)

            @pl.loop(0, C // 2)
            def _row(r, bb=bb):
                for j in range(D // LANES):
                    sl = pl.ds(j * LANES, LANES)
                    sr = pl.ds(D + j * LANES, LANES)
                    pk_v[bb, r, sl] = ge_v[bb, r, sl] * SCALE
                    pk_v[bb, r, sr] = go_v[bb, r, sl] * SCALE

            scatter(g, bb, sem_s[bb]).start()

    for b in range(NBUF):
        scatter(n_chunks - NBUF + b, b, sem_s[b]).wait()


def kernel(x, table):
    batch, seq = x.shape
    b_total = batch * seq
    n_chunks = b_total // (NW * C)
    xf = x.astype(jnp.int32).reshape(b_total // 2, 2)
    xe = xf[:, 0].reshape(b_total // C, C // 2)
    xo = xf[:, 1].reshape(b_total // C, C // 2)

    mesh = plsc.VectorSubcoreMesh(
        core_axis_name="c", subcore_axis_name="s", num_cores=NC,
        num_subcores=NS)
    out = pl.kernel(
        _emb_body,
        out_type=jax.ShapeDtypeStruct((b_total // 2, 2 * D), jnp.float32),
        mesh=mesh,
        scratch_types=[
            pltpu.VMEM((2, n_chunks, C // 2), jnp.int32),
            pltpu.VMEM((NBUF, C // 2, D), jnp.float32),
            pltpu.VMEM((NBUF, C // 2, D), jnp.float32),
            pltpu.VMEM((NBUF, C // 2, 2 * D), jnp.float32),
            *([pltpu.SemaphoreType.DMA] * (3 * NBUF)),
        ],
        compiler_params=pltpu.CompilerParams(use_tc_tiling_on_sc=False),
    )(xe, xo, table)
    return out.reshape(batch, seq, D)


# speedup vs baseline: 1.1628x; 1.1628x over previous
import math

import jax
import jax.numpy as jnp
from jax import lax
from jax.experimental import pallas as pl
from jax.experimental.pallas import tpu as pltpu
from jax.experimental.pallas import tpu_sc as plsc

D = 64
SCALE = math.sqrt(D)
NC = 2
NS = 16
NW = NC * NS
H = 100
NBUF = 4
LANES = 16


def _emb_body(x_hbm, table_hbm, out_hbm, idx_v, rows_v, pk_v, *sems):
    sem_g0 = sems[:NBUF]
    sem_g1 = sems[NBUF:2 * NBUF]
    sem_s = sems[2 * NBUF:]
    seq = out_hbm.shape[1]
    rows_per_w = out_hbm.shape[0] // NW
    n_chunks = 2 * rows_per_w
    wid = lax.axis_index("s") * NC + lax.axis_index("c")
    base = wid * rows_per_w

    def gather_start(g, b):
        pltpu.make_async_copy(
            table_hbm.at[idx_v.at[g]], rows_v.at[b], sem_g0[b]).start()

    def gather_wait(g, b):
        pltpu.make_async_copy(
            table_hbm.at[idx_v.at[g]], rows_v.at[b], sem_g0[b]).wait()

    def scatter(g, b, sem):
        return pltpu.make_async_copy(
            pk_v.at[b],
            out_hbm.at[base + g // 2, pl.ds((g % 2) * H, H)], sem)

    pltpu.sync_copy(x_hbm.at[pl.ds(wid * 2 * rows_per_w, 2 * rows_per_w)],
                    idx_v)

    for b in range(NBUF):
        gather_start(b, b)

    @pl.loop(0, n_chunks // NBUF)
    def _outer(t):
        g0 = t * NBUF
        for bb in range(NBUF):
            g = g0 + bb
            pb = (bb - 1) % NBUF
            p = g - 1
            nxt = p + NBUF

            @pl.when(jnp.logical_and(p >= 0, nxt < n_chunks))
            def _recycle(pb=pb, p=p, nxt=nxt):
                scatter(p, pb, sem_s[pb]).wait()
                gather_start(nxt, pb)

            gather_wait(g, bb)

            @pl.loop(0, H)
            def _row(r, bb=bb):
                for j in range(D // LANES):
                    sl = pl.ds(j * LANES, LANES)
                    pk_v[bb, r, sl] = rows_v[bb, r, sl] * SCALE

            scatter(g, bb, sem_s[bb]).start()

    for b in range(NBUF):
        scatter(n_chunks - NBUF + b, b, sem_s[b]).wait()


def kernel(x, table):
    batch, seq = x.shape
    x2 = x.astype(jnp.int32).reshape(batch * seq // H, H)
    rows_per_w = batch // NW
    tp = jnp.pad(table, ((0, 0), (0, 128 - D)))

    mesh = plsc.VectorSubcoreMesh(
        core_axis_name="c", subcore_axis_name="s", num_cores=NC,
        num_subcores=NS)
    out = pl.kernel(
        _emb_body,
        out_type=jax.ShapeDtypeStruct((batch, seq, D), jnp.float32),
        mesh=mesh,
        scratch_types=[
            pltpu.VMEM((2 * rows_per_w, H), jnp.int32),
            pltpu.VMEM((NBUF, H, 2 * D), jnp.float32),
            pltpu.VMEM((NBUF, H, D), jnp.float32),
            *([pltpu.SemaphoreType.DMA] * (3 * NBUF)),
        ],
        compiler_params=pltpu.CompilerParams(use_tc_tiling_on_sc=False),
    )(x2, tp)
    return out
